# Initial kernel scaffold; baseline (speedup 1.0000x reference)
import jax
import jax.numpy as jnp
from jax.experimental import pallas as pl


def kernel(pos, batch):
    raise NotImplementedError("write your pallas kernel here")


# speedup vs baseline: 6.8469x; 6.8469x over previous
import functools

import jax
import jax.numpy as jnp
from jax import lax
from jax.experimental import pallas as pl
from jax.experimental.pallas import tpu as pltpu
from jax.experimental.pallas import tpu_sc as plsc

_B = 16
_M = 4096
_NS = 1024
_L = 16
_C = _M // _L


def _build_fps():
    mesh = plsc.VectorSubcoreMesh(core_axis_name="c", subcore_axis_name="s")

    @functools.partial(
        pl.kernel,
        out_type=jax.ShapeDtypeStruct((_B, _NS), jnp.int32),
        mesh=mesh,
        compiler_params=pltpu.CompilerParams(needs_layout_passes=False),
        scratch_types=[
            pltpu.VMEM((_M,), jnp.float32),
            pltpu.VMEM((_M,), jnp.float32),
            pltpu.VMEM((_M,), jnp.float32),
            pltpu.VMEM((_M,), jnp.float32),
            pltpu.VMEM((_NS,), jnp.int32),
            pltpu.VMEM((_L,), jnp.int32),
        ],
    )
    def fps(xs_hbm, ys_hbm, zs_hbm, out_hbm, x_v, y_v, z_v, d_v, o_v, mi_v):
        cid = lax.axis_index("c")
        sid = lax.axis_index("s")
        wid = sid * 2 + cid

        @pl.when(wid < _B)
        def _():
            b = wid
            pltpu.sync_copy(xs_hbm.at[b], x_v)
            pltpu.sync_copy(ys_hbm.at[b], y_v)
            pltpu.sync_copy(zs_hbm.at[b], z_v)
            lanes = lax.iota(jnp.int32, _L)
            zeros = jnp.zeros((_L,), jnp.int32)
            lane0 = lanes == 0

            def coords(nxt):
                return (
                    plsc.load_gather(x_v, [nxt]),
                    plsc.load_gather(y_v, [nxt]),
                    plsc.load_gather(z_v, [nxt]),
                )

            def one_pass(ax, ay, az, first):
                neg = jnp.full((_L,), -jnp.inf, jnp.float32)

                def body(k, carry):
                    mv, mi = carry
                    idxv = k * _L + lanes
                    dx = plsc.load_gather(x_v, [idxv]) - ax
                    dy = plsc.load_gather(y_v, [idxv]) - ay
                    dz = plsc.load_gather(z_v, [idxv]) - az
                    nd = dx * dx + dy * dy + dz * dz
                    if not first:
                        nd = jnp.minimum(plsc.load_gather(d_v, [idxv]), nd)
                    plsc.store_scatter(d_v, [idxv], nd)
                    pred = nd > mv
                    mv = jnp.where(pred, nd, mv)
                    mi = jnp.where(pred, idxv, mi)
                    return mv, mi

                mv, mi = lax.fori_loop(0, _C, body, (neg, zeros))
                m = jnp.max(mv)
                lane = plsc.all_reduce_ffs(mv == m)
                mi_v[...] = mi
                return plsc.load_gather(mi_v, [lane])

            plsc.store_scatter(o_v, [zeros], zeros, mask=lane0)
            v0x = plsc.load_gather(x_v, [lanes])
            v0y = plsc.load_gather(y_v, [lanes])
            v0z = plsc.load_gather(z_v, [lanes])
            a0 = (
                jnp.broadcast_to(jnp.sum(jnp.where(lane0, v0x, 0.0)), (_L,)),
                jnp.broadcast_to(jnp.sum(jnp.where(lane0, v0y, 0.0)), (_L,)),
                jnp.broadcast_to(jnp.sum(jnp.where(lane0, v0z, 0.0)), (_L,)),
            )
            nxt = one_pass(*a0, first=True)
            plsc.store_scatter(o_v, [jnp.full((_L,), 1, jnp.int32)], nxt, mask=lane0)

            def step(i, carry):
                nxt_i = one_pass(*carry, first=False)
                pos_i = jnp.broadcast_to(i, (_L,)).astype(jnp.int32)
                plsc.store_scatter(o_v, [pos_i], nxt_i, mask=lane0)
                return coords(nxt_i)

            lax.fori_loop(2, _NS, step, coords(nxt))
            pltpu.sync_copy(o_v, out_hbm.at[b])

    return fps


_fps = _build_fps()


def kernel(pos, batch):
    del batch
    p = pos.reshape(_B, _M, 3)
    idx = _fps(p[:, :, 0], p[:, :, 1], p[:, :, 2])
    base = jnp.arange(_B, dtype=jnp.int32)[:, None] * _M
    return (base + idx).reshape(-1)
